# baseline (speedup 1.0000x reference)
import functools

import jax
import jax.numpy as jnp
from jax import lax
from jax.experimental import pallas as pl
from jax.experimental.pallas import tpu as pltpu
from jax.experimental.pallas import tpu_sc as plsc

NUM_USERS = 1000000
BATCH = 16384
HIDDEN = 16
NUM_CORES = 2
NUM_SUBCORES = 16
NUM_WORKERS = NUM_CORES * NUM_SUBCORES
BPW = BATCH // NUM_WORKERS
LANES = 16
CHUNK = 8
NPAIRS = BPW // (2 * CHUNK)
IDX_PAD = BPW + LANES


def _gmf_body(uidx_hbm, iidx_hbm, utab_hbm, itab_hbm, w_hbm, b_hbm, out_hbm,
              uidx_v, iidx_v, ustage_a, istage_a, ustage_b, istage_b,
              w_v, b_v, out_v, sem_a, sem_b):
    wid = lax.axis_index("s") * NUM_CORES + lax.axis_index("c")
    base = wid * BPW

    pltpu.sync_copy(uidx_hbm.at[pl.ds(base, BPW)], uidx_v.at[pl.ds(0, BPW)])
    pltpu.sync_copy(iidx_hbm.at[pl.ds(base, BPW)], iidx_v.at[pl.ds(0, BPW)])
    pltpu.sync_copy(w_hbm, w_v)
    pltpu.sync_copy(b_hbm, b_v)

    wvec = w_v[...]
    bvec = b_v[...]
    iota = lax.iota(jnp.int32, LANES)
    masks = [iota == k for k in range(LANES)]
    zeros = jnp.zeros((LANES,), jnp.float32)
    zeros_i = jnp.zeros((LANES,), jnp.int32)

    def fire(c, ustage, istage, sem):
        uvec = uidx_v[pl.ds(c * CHUNK, LANES)]
        ivec = iidx_v[pl.ds(c * CHUNK, LANES)]
        ualign = (uvec >> 7) << 7
        ialign = (ivec >> 7) << 7
        for k in range(CHUNK):
            uoff = pl.multiple_of(ualign[k], 128)
            ioff = pl.multiple_of(ialign[k], 128)
            if k % 2 == 0:
                pltpu.make_async_copy(
                    utab_hbm.at[iota, pl.ds(uoff, 128)], ustage.at[k], sem
                ).start()
                pltpu.make_async_copy(
                    itab_hbm.at[iota, pl.ds(ioff, 128)], istage.at[k], sem
                ).start()
            else:
                pltpu.make_async_copy(
                    utab_hbm.at[pl.ds(0, HIDDEN), pl.ds(uoff, 128)],
                    ustage.at[k], sem
                ).start()
                pltpu.make_async_copy(
                    itab_hbm.at[pl.ds(0, HIDDEN), pl.ds(ioff, 128)],
                    istage.at[k], sem
                ).start()

    def drain(ustage, istage, sem):
        for k in range(CHUNK):
            if k % 2 == 0:
                pltpu.make_async_copy(
                    utab_hbm.at[iota, pl.ds(0, 128)], ustage.at[k], sem
                ).wait()
                pltpu.make_async_copy(
                    itab_hbm.at[iota, pl.ds(0, 128)], istage.at[k], sem
                ).wait()
            else:
                pltpu.make_async_copy(
                    utab_hbm.at[pl.ds(0, HIDDEN), pl.ds(0, 128)],
                    ustage.at[k], sem
                ).wait()
                pltpu.make_async_copy(
                    itab_hbm.at[pl.ds(0, HIDDEN), pl.ds(0, 128)],
                    istage.at[k], sem
                ).wait()

    def accum(c, ustage, istage, acc, k0):
        uvec = uidx_v[pl.ds(c * CHUNK, LANES)]
        ivec = iidx_v[pl.ds(c * CHUNK, LANES)]
        ulane = uvec & 127
        ilane = ivec & 127
        for k in range(CHUNK):
            urow = plsc.load_gather(
                ustage, [zeros_i + k, iota, zeros_i + ulane[k]]
            )
            irow = plsc.load_gather(
                istage, [zeros_i + k, iota, zeros_i + ilane[k]]
            )
            p = urow * irow * wvec
            acc = jnp.where(masks[k0 + k], jnp.sum(p), acc)
        return acc

    fire(0, ustage_a, istage_a, sem_a)

    def pair(c2, carry):
        c = 2 * c2
        fire(c + 1, ustage_b, istage_b, sem_b)
        drain(ustage_a, istage_a, sem_a)
        acc = accum(c, ustage_a, istage_a, zeros, 0)

        @pl.when(c2 < NPAIRS - 1)
        def _():
            fire(c + 2, ustage_a, istage_a, sem_a)

        drain(ustage_b, istage_b, sem_b)
        acc = accum(c + 1, ustage_b, istage_b, acc, CHUNK)
        z = acc + bvec
        out_v[pl.ds(c * CHUNK, LANES)] = 1.0 / (1.0 + jnp.exp(-z))
        return carry

    lax.fori_loop(0, NPAIRS, pair, 0)

    pltpu.sync_copy(out_v, out_hbm.at[pl.ds(base, BPW)])


@functools.partial(
    pl.kernel,
    mesh=plsc.VectorSubcoreMesh(core_axis_name="c", subcore_axis_name="s"),
    out_type=jax.ShapeDtypeStruct((BATCH,), jnp.float32),
    compiler_params=pltpu.CompilerParams(needs_layout_passes=False),
    scratch_types=[
        pltpu.VMEM((IDX_PAD,), jnp.int32),
        pltpu.VMEM((IDX_PAD,), jnp.int32),
        pltpu.VMEM((CHUNK, HIDDEN, 128), jnp.float32),
        pltpu.VMEM((CHUNK, HIDDEN, 128), jnp.float32),
        pltpu.VMEM((CHUNK, HIDDEN, 128), jnp.float32),
        pltpu.VMEM((CHUNK, HIDDEN, 128), jnp.float32),
        pltpu.VMEM((LANES,), jnp.float32),
        pltpu.VMEM((LANES,), jnp.float32),
        pltpu.VMEM((BPW,), jnp.float32),
        pltpu.SemaphoreType.DMA,
        pltpu.SemaphoreType.DMA,
    ],
)
def _gmf_sc(*refs):
    _gmf_body(*refs)


def kernel(user_indices, item_indices, user_table, item_table, W, b):
    uidx = user_indices.astype(jnp.int32)
    iidx = item_indices.astype(jnp.int32)
    ut2 = user_table.T
    it2 = item_table.T
    w16 = W.astype(jnp.float32).reshape(HIDDEN)
    b16 = jnp.broadcast_to(b.astype(jnp.float32), (LANES,))
    out = _gmf_sc(uidx, iidx, ut2, it2, w16, b16)
    return out.reshape(BATCH, 1)


# speedup vs baseline: 1.0663x; 1.0663x over previous
import functools

import jax
import jax.numpy as jnp
from jax import lax
from jax.experimental import pallas as pl
from jax.experimental.pallas import tpu as pltpu
from jax.experimental.pallas import tpu_sc as plsc

NUM_USERS = 1000000
BATCH = 16384
HIDDEN = 16
NUM_CORES = 2
NUM_SUBCORES = 16
NUM_WORKERS = NUM_CORES * NUM_SUBCORES
BPW = BATCH // NUM_WORKERS
LANES = 16
CHUNK = 8
NPAIRS = BPW // (2 * CHUNK)
IDX_PAD = BPW + LANES


def _gmf_body(uidx_hbm, iidx_hbm, utab_hbm, itab_hbm, w_hbm, b_hbm, out_hbm,
              uidx_v, iidx_v, ustage_a, istage_a, ustage_b, istage_b,
              w_v, b_v, out_v, sem_a, sem_b):
    wid = lax.axis_index("s") * NUM_CORES + lax.axis_index("c")
    base = wid * BPW

    pltpu.sync_copy(uidx_hbm.at[pl.ds(base, BPW)], uidx_v.at[pl.ds(0, BPW)])
    pltpu.sync_copy(iidx_hbm.at[pl.ds(base, BPW)], iidx_v.at[pl.ds(0, BPW)])
    pltpu.sync_copy(w_hbm, w_v)
    pltpu.sync_copy(b_hbm, b_v)

    wvec = w_v[...]
    bvec = b_v[...]
    iota = lax.iota(jnp.int32, LANES)
    masks = [iota == k for k in range(LANES)]
    zeros = jnp.zeros((LANES,), jnp.float32)
    zeros_i = jnp.zeros((LANES,), jnp.int32)

    def fire(c, ustage, istage, sem):
        uvec = uidx_v[pl.ds(c * CHUNK, LANES)]
        ivec = iidx_v[pl.ds(c * CHUNK, LANES)]
        ualign = (uvec >> 7) << 7
        ialign = (ivec >> 7) << 7
        for k in range(CHUNK):
            uoff = pl.multiple_of(ualign[k], 128)
            ioff = pl.multiple_of(ialign[k], 128)
            pltpu.make_async_copy(
                utab_hbm.at[iota, pl.ds(uoff, 128)], ustage.at[k], sem
            ).start()
            pltpu.make_async_copy(
                itab_hbm.at[iota, pl.ds(ioff, 128)], istage.at[k], sem
            ).start()

    def drain(ustage, istage, sem):
        for k in range(CHUNK):
            pltpu.make_async_copy(
                utab_hbm.at[iota, pl.ds(0, 128)], ustage.at[k], sem
            ).wait()
            pltpu.make_async_copy(
                itab_hbm.at[iota, pl.ds(0, 128)], istage.at[k], sem
            ).wait()

    def accum(c, ustage, istage, acc, k0):
        uvec = uidx_v[pl.ds(c * CHUNK, LANES)]
        ivec = iidx_v[pl.ds(c * CHUNK, LANES)]
        ulane = uvec & 127
        ilane = ivec & 127
        for k in range(CHUNK):
            urow = plsc.load_gather(
                ustage, [zeros_i + k, iota, zeros_i + ulane[k]]
            )
            irow = plsc.load_gather(
                istage, [zeros_i + k, iota, zeros_i + ilane[k]]
            )
            p = urow * irow * wvec
            acc = jnp.where(masks[k0 + k], jnp.sum(p), acc)
        return acc

    fire(0, ustage_a, istage_a, sem_a)

    def pair(c2, carry):
        c = 2 * c2
        fire(c + 1, ustage_b, istage_b, sem_b)
        drain(ustage_a, istage_a, sem_a)
        acc = accum(c, ustage_a, istage_a, zeros, 0)

        @pl.when(c2 < NPAIRS - 1)
        def _():
            fire(c + 2, ustage_a, istage_a, sem_a)

        drain(ustage_b, istage_b, sem_b)
        acc = accum(c + 1, ustage_b, istage_b, acc, CHUNK)
        z = acc + bvec
        out_v[pl.ds(c * CHUNK, LANES)] = 1.0 / (1.0 + jnp.exp(-z))
        return carry

    lax.fori_loop(0, NPAIRS, pair, 0)

    pltpu.sync_copy(out_v, out_hbm.at[pl.ds(base, BPW)])


@functools.partial(
    pl.kernel,
    mesh=plsc.VectorSubcoreMesh(core_axis_name="c", subcore_axis_name="s"),
    out_type=jax.ShapeDtypeStruct((BATCH,), jnp.float32),
    compiler_params=pltpu.CompilerParams(needs_layout_passes=False),
    scratch_types=[
        pltpu.VMEM((IDX_PAD,), jnp.int32),
        pltpu.VMEM((IDX_PAD,), jnp.int32),
        pltpu.VMEM((CHUNK, HIDDEN, 128), jnp.float32),
        pltpu.VMEM((CHUNK, HIDDEN, 128), jnp.float32),
        pltpu.VMEM((CHUNK, HIDDEN, 128), jnp.float32),
        pltpu.VMEM((CHUNK, HIDDEN, 128), jnp.float32),
        pltpu.VMEM((LANES,), jnp.float32),
        pltpu.VMEM((LANES,), jnp.float32),
        pltpu.VMEM((BPW,), jnp.float32),
        pltpu.SemaphoreType.DMA,
        pltpu.SemaphoreType.DMA,
    ],
)
def _gmf_sc(*refs):
    _gmf_body(*refs)


def kernel(user_indices, item_indices, user_table, item_table, W, b):
    uidx = user_indices.astype(jnp.int32)
    iidx = item_indices.astype(jnp.int32)
    ut2 = user_table.T
    it2 = item_table.T
    w16 = W.astype(jnp.float32).reshape(HIDDEN)
    b16 = jnp.broadcast_to(b.astype(jnp.float32), (LANES,))
    out = _gmf_sc(uidx, iidx, ut2, it2, w16, b16)
    return out.reshape(BATCH, 1)
